# baseline (speedup 1.0000x reference)
import functools

import jax
import jax.numpy as jnp
from jax import lax
from jax.experimental import pallas as pl
from jax.experimental.pallas import tpu as pltpu
from jax.experimental.pallas import tpu_sc as plsc

V = 5000
VP = 5120
F = 10000
PAD_COORD = 1.0e6
TI = 512
NW = 32
FPW = 320
QPW = VP // NW
D2_THRESH = 0.005 * 0.005

_SC_MESH = plsc.VectorSubcoreMesh(core_axis_name="c", subcore_axis_name="s")
_SC_PARAMS = pltpu.CompilerParams(
    use_tc_tiling_on_sc=False, needs_layout_passes=False)


def _normals_body(mesh_hbm, recon_hbm, face_hbm, rface_hbm, out_hbm,
                  vtab, fidxm, fidxr, acc, sem):
    cid = lax.axis_index("c")
    sid = lax.axis_index("s")
    wid = sid * 2 + cid
    B = mesh_hbm.shape[0]
    lane = lax.iota(jnp.int32, 16)
    c0 = jnp.zeros((16,), jnp.int32)
    c1 = c0 + 1
    c2 = c0 + 2
    off = jnp.minimum(wid * FPW, F - FPW)
    pltpu.sync_copy(face_hbm.at[pl.ds(off, FPW)], fidxm)

    def run_table(verts_src, b, fidx, t):
        cp = pltpu.async_copy(verts_src.at[b], vtab, sem)

        def zbody(k, carry):
            acc[pl.ds(k * 16, 16)] = jnp.zeros((16,), jnp.float32)
            return carry
        lax.fori_loop(0, (3 * VP) // 16, zbody, 0)
        cp.wait()

        def fbody(k, carry):
            base = k * 16
            gface = off + base + lane
            own = (gface >= wid * FPW) & (gface < F)
            row = base + lane
            i0 = plsc.load_gather(fidx, [row, c0])
            i1 = plsc.load_gather(fidx, [row, c1])
            i2 = plsc.load_gather(fidx, [row, c2])

            def g(ix, c):
                return plsc.load_gather(vtab, [ix, c])
            v0x, v0y, v0z = g(i0, c0), g(i0, c1), g(i0, c2)
            e1x = g(i1, c0) - v0x
            e1y = g(i1, c1) - v0y
            e1z = g(i1, c2) - v0z
            e2x = g(i2, c0) - v0x
            e2y = g(i2, c1) - v0y
            e2z = g(i2, c2) - v0z
            cx = e1y * e2z - e1z * e2y
            cy = e1z * e2x - e1x * e2z
            cz = e1x * e2y - e1y * e2x
            for ix in (i0, i1, i2):
                plsc.addupdate_scatter(acc, [ix], cx, mask=own)
                plsc.addupdate_scatter(acc, [ix + VP], cy, mask=own)
                plsc.addupdate_scatter(acc, [ix + 2 * VP], cz, mask=own)
            return carry
        lax.fori_loop(0, FPW // 16, fbody, 0)
        pltpu.sync_copy(acc, out_hbm.at[t, wid])

    for b in range(B):
        run_table(mesh_hbm, b, fidxm, b)
        pltpu.sync_copy(rface_hbm.at[b, pl.ds(off, FPW)], fidxr)
        run_table(recon_hbm, b, fidxr, B + b)


def _normals_partials(meshes, recons, face, rfaces):
    B = meshes.shape[0]
    run = pl.kernel(
        _normals_body,
        out_type=jax.ShapeDtypeStruct((2 * B, NW, 3 * VP), jnp.float32),
        mesh=_SC_MESH,
        compiler_params=_SC_PARAMS,
        scratch_types=[
            pltpu.VMEM((V, 3), jnp.float32),
            pltpu.VMEM((FPW, 3), jnp.int32),
            pltpu.VMEM((FPW, 3), jnp.int32),
            pltpu.VMEM((3 * VP,), jnp.float32),
            pltpu.SemaphoreType.DMA,
        ],
    )
    return run(meshes, recons, face, rfaces)


def _knn_body(q_ref, tT_ref, rowd_ref, rowi_ref, cold_ref, coli_ref):
    i = pl.program_id(1)
    q = q_ref[...]
    tT = tT_ref[0]
    dx = q[:, 0:1] - tT[0:1, :]
    dy = q[:, 1:2] - tT[1:2, :]
    dz = q[:, 2:3] - tT[2:3, :]
    d2 = dx * dx + dy * dy + dz * dz
    iota_j = lax.broadcasted_iota(jnp.int32, (TI, VP), 1)
    iota_i = lax.broadcasted_iota(jnp.int32, (TI, VP), 0) + i * TI
    rmin = jnp.min(d2, axis=1)
    rarg = jnp.min(jnp.where(d2 == rmin[:, None], iota_j, 2**30), axis=1)
    rowd_ref[0, pl.ds(i * TI, TI)] = rmin
    rowi_ref[0, pl.ds(i * TI, TI)] = rarg
    cmin = jnp.min(d2, axis=0)
    carg = jnp.min(jnp.where(d2 == cmin[None, :], iota_i, 2**30), axis=0)

    @pl.when(i == 0)
    def _():
        cold_ref[0] = cmin
        coli_ref[0] = carg

    @pl.when(i > 0)
    def _():
        upd = cmin < cold_ref[0]
        cold_ref[0] = jnp.where(upd, cmin, cold_ref[0])
        coli_ref[0] = jnp.where(upd, carg, coli_ref[0])


def _knn_both(meshes_p, recons_pT):
    B = meshes_p.shape[0]
    ni = VP // TI
    grid = (B, ni)
    outs = pl.pallas_call(
        _knn_body,
        grid=grid,
        in_specs=[
            pl.BlockSpec((None, TI, 3), lambda b, i: (b, i, 0)),
            pl.BlockSpec((1, 3, VP), lambda b, i: (b, 0, 0)),
        ],
        out_specs=[
            pl.BlockSpec((None, 1, VP), lambda b, i: (b, 0, 0)),
            pl.BlockSpec((None, 1, VP), lambda b, i: (b, 0, 0)),
            pl.BlockSpec((None, 1, VP), lambda b, i: (b, 0, 0)),
            pl.BlockSpec((None, 1, VP), lambda b, i: (b, 0, 0)),
        ],
        out_shape=[
            jax.ShapeDtypeStruct((B, 1, VP), jnp.float32),
            jax.ShapeDtypeStruct((B, 1, VP), jnp.int32),
            jax.ShapeDtypeStruct((B, 1, VP), jnp.float32),
            jax.ShapeDtypeStruct((B, 1, VP), jnp.int32),
        ],
    )(meshes_p, recons_pT)
    rowd, rowi, cold, coli = outs
    return (rowd.reshape(B, VP), rowi.reshape(B, VP),
            cold.reshape(B, VP), coli.reshape(B, VP))


def _norm_body(p_ref, n_ref):
    vn = jnp.sum(p_ref[0], axis=0)
    norm = jnp.sqrt(jnp.sum(vn * vn, axis=0, keepdims=True))
    n_ref[0] = vn / jnp.maximum(norm, 1e-12)


def _normalize(partials):
    nt = partials.shape[0]
    return pl.pallas_call(
        _norm_body,
        grid=(nt,),
        in_specs=[pl.BlockSpec((1, NW, 3, VP), lambda t: (t, 0, 0, 0))],
        out_specs=pl.BlockSpec((1, 3, VP), lambda t: (t, 0, 0)),
        out_shape=jax.ShapeDtypeStruct((nt, 3, VP), jnp.float32),
    )(partials)


def _loss_body(mesh_hbm, recon_hbm, mn_hbm, rn_hbm,
               ir_hbm, dr_hbm, ic_hbm, dc_hbm, out_hbm,
               mtab, rtab, rntab, mnbuf, irbuf, drbuf, icbuf, dcbuf, outbuf,
               sem):
    cid = lax.axis_index("c")
    sid = lax.axis_index("s")
    wid = sid * 2 + cid
    B = mesh_hbm.shape[0]
    lane = lax.iota(jnp.int32, 16)
    zeros = jnp.zeros((16,), jnp.float32)
    for b in range(B):
        cps = [
            pltpu.async_copy(mesh_hbm.at[b], mtab, sem),
            pltpu.async_copy(recon_hbm.at[b], rtab, sem),
            pltpu.async_copy(rn_hbm.at[b], rntab, sem),
            pltpu.async_copy(mn_hbm.at[b, :, pl.ds(wid * QPW, QPW)], mnbuf,
                             sem),
            pltpu.async_copy(ir_hbm.at[b, pl.ds(wid * QPW, QPW)], irbuf, sem),
            pltpu.async_copy(dr_hbm.at[b, pl.ds(wid * QPW, QPW)], drbuf, sem),
            pltpu.async_copy(ic_hbm.at[b, pl.ds(wid * QPW, QPW)], icbuf, sem),
            pltpu.async_copy(dc_hbm.at[b, pl.ds(wid * QPW, QPW)], dcbuf, sem),
        ]
        for cp in cps:
            cp.wait()

        def kbody(k, carry):
            a0, a1, a2 = carry
            base = k * 16
            q3 = (wid * QPW + base + lane) * 3
            valid = (wid * QPW + base + lane) < V
            idx = irbuf[pl.ds(base, 16)]
            idx3 = idx * 3
            d2r = drbuf[pl.ds(base, 16)]
            nnx = plsc.load_gather(rtab, [idx3])
            nny = plsc.load_gather(rtab, [idx3 + 1])
            nnz = plsc.load_gather(rtab, [idx3 + 2])
            rnx = plsc.load_gather(rntab, [idx])
            rny = plsc.load_gather(rntab, [idx + VP])
            rnz = plsc.load_gather(rntab, [idx + 2 * VP])
            mx = plsc.load_gather(mtab, [q3])
            my = plsc.load_gather(mtab, [q3 + 1])
            mz = plsc.load_gather(mtab, [q3 + 2])
            mnx = mnbuf[0, pl.ds(base, 16)]
            mny = mnbuf[1, pl.ds(base, 16)]
            mnz = mnbuf[2, pl.ds(base, 16)]
            ndot = mnx * rnx + mny * rny + mnz * rnz
            ok1 = valid & (ndot >= 0.45) & (d2r <= D2_THRESH)
            w1 = jnp.where(ok1, 1.0, 0.0)
            a0 = a0 + jnp.abs((mx - nnx) * rnx) * w1
            a1 = a1 + jnp.abs((my - nny) * rny) * w1
            a2 = a2 + jnp.abs((mz - nnz) * rnz) * w1
            idc = icbuf[pl.ds(base, 16)]
            idc3 = idc * 3
            d2c = dcbuf[pl.ds(base, 16)]
            gx = plsc.load_gather(mtab, [idc3])
            gy = plsc.load_gather(mtab, [idc3 + 1])
            gz = plsc.load_gather(mtab, [idc3 + 2])
            rx = plsc.load_gather(rtab, [q3])
            ry = plsc.load_gather(rtab, [q3 + 1])
            rz = plsc.load_gather(rtab, [q3 + 2])
            w2 = jnp.where(valid & (d2c <= D2_THRESH), 1.0, 0.0)
            a0 = a0 + jnp.abs(rx - gx) * w2
            a1 = a1 + jnp.abs(ry - gy) * w2
            a2 = a2 + jnp.abs(rz - gz) * w2
            return (a0, a1, a2)

        a0, a1, a2 = lax.fori_loop(0, QPW // 16, kbody, (zeros, zeros, zeros))
        outbuf[0] = a0
        outbuf[1] = a1
        outbuf[2] = a2
        outbuf[3] = zeros
        pltpu.sync_copy(outbuf, out_hbm.at[b, wid])


def _loss_partials(mesh_i, recon_i, mesh_nT, rec_nT, ir, dr, ic, dc):
    B = mesh_i.shape[0]
    run = pl.kernel(
        _loss_body,
        out_type=jax.ShapeDtypeStruct((B, NW, 4, 16), jnp.float32),
        mesh=_SC_MESH,
        compiler_params=_SC_PARAMS,
        scratch_types=[
            pltpu.VMEM((3 * VP,), jnp.float32),
            pltpu.VMEM((3 * VP,), jnp.float32),
            pltpu.VMEM((3 * VP,), jnp.float32),
            pltpu.VMEM((3, QPW), jnp.float32),
            pltpu.VMEM((QPW,), jnp.int32),
            pltpu.VMEM((QPW,), jnp.float32),
            pltpu.VMEM((QPW,), jnp.int32),
            pltpu.VMEM((QPW,), jnp.float32),
            pltpu.VMEM((4, 16), jnp.float32),
            pltpu.SemaphoreType.DMA,
        ],
    )
    return run(mesh_i, recon_i, mesh_nT, rec_nT.reshape(B, 3 * VP),
               ir, dr, ic, dc)


def _fin_body(p_ref, o_ref):
    x = p_ref[0]
    s = jnp.sum(jnp.sum(x, axis=0), axis=1)
    o_ref[0] = s * (1.0 / V)


def _finalize(psums):
    B = psums.shape[0]
    out = pl.pallas_call(
        _fin_body,
        grid=(B,),
        in_specs=[pl.BlockSpec((1, NW, 4, 16), lambda b: (b, 0, 0, 0))],
        out_specs=pl.BlockSpec((None, 1, 4), lambda b: (b, 0, 0)),
        out_shape=jax.ShapeDtypeStruct((B, 1, 4), jnp.float32),
    )(psums)
    return out[:, 0, :3]


@jax.jit
def _impl(joints, meshes, recons, face, recons_faces):
    B = meshes.shape[0]
    pad = ((0, 0), (0, VP - V), (0, 0))
    meshes_p = jnp.pad(meshes, pad, constant_values=PAD_COORD)
    recons_p = jnp.pad(recons, pad, constant_values=PAD_COORD)
    recons_pT = jnp.transpose(recons_p, (0, 2, 1))

    partials = _normals_partials(meshes, recons, face, recons_faces)
    normals = _normalize(partials.reshape(2 * B, NW, 3, VP))
    mesh_nT, rec_nT = normals[:B], normals[B:]

    row_d2, row_idx, col_d2, col_idx = _knn_both(meshes_p, recons_pT)

    psums = _loss_partials(meshes_p.reshape(B, 3 * VP),
                           recons_p.reshape(B, 3 * VP), mesh_nT, rec_nT,
                           row_idx, row_d2, col_idx, col_d2)
    return _finalize(psums)


def kernel(joints, meshes, recons, face, recons_faces):
    return _impl(joints, meshes, recons, face, recons_faces)


# speedup vs baseline: 1.1319x; 1.1319x over previous
import functools

import jax
import jax.numpy as jnp
from jax import lax
from jax.experimental import pallas as pl
from jax.experimental.pallas import tpu as pltpu
from jax.experimental.pallas import tpu_sc as plsc

V = 5000
VP = 5120
F = 10000
PAD_COORD = 1.0e6
TI = 512
NW = 32
FPW = 320
FP = NW * FPW
QPW = VP // NW
D2_THRESH = 0.005 * 0.005

_SC_MESH = plsc.VectorSubcoreMesh(core_axis_name="c", subcore_axis_name="s")
_SC_PARAMS = pltpu.CompilerParams(
    use_tc_tiling_on_sc=False, needs_layout_passes=False)


def _normals_body(mesh_hbm, recon_hbm, face_hbm, rface_hbm, out_hbm,
                  vtab, fidxm, fidxr, acc, sem):
    cid = lax.axis_index("c")
    sid = lax.axis_index("s")
    wid = sid * 2 + cid
    B = mesh_hbm.shape[0]
    lane = lax.iota(jnp.int32, 16)
    pltpu.sync_copy(face_hbm.at[pl.ds(wid * FPW * 3, FPW * 3)], fidxm)

    def run_table(verts_src, b, fidx, t):
        cp = pltpu.async_copy(verts_src.at[b], vtab, sem)

        def zbody(k, carry):
            acc[pl.ds(k * 16, 16)] = jnp.zeros((16,), jnp.float32)
            return carry
        lax.fori_loop(0, (3 * VP) // 16, zbody, 0)
        cp.wait()

        def fbody(k, carry):
            base = k * 16
            valid = (wid * FPW + base + lane) < F
            idx3 = (base + lane) * 3
            i0 = plsc.load_gather(fidx, [idx3])
            i1 = plsc.load_gather(fidx, [idx3 + 1])
            i2 = plsc.load_gather(fidx, [idx3 + 2])

            def g(ix, c):
                return plsc.load_gather(vtab, [ix * 3 + c])
            v0x, v0y, v0z = g(i0, 0), g(i0, 1), g(i0, 2)
            e1x = g(i1, 0) - v0x
            e1y = g(i1, 1) - v0y
            e1z = g(i1, 2) - v0z
            e2x = g(i2, 0) - v0x
            e2y = g(i2, 1) - v0y
            e2z = g(i2, 2) - v0z
            cx = e1y * e2z - e1z * e2y
            cy = e1z * e2x - e1x * e2z
            cz = e1x * e2y - e1y * e2x
            for ix in (i0, i1, i2):
                plsc.addupdate_scatter(acc, [ix], cx, mask=valid)
                plsc.addupdate_scatter(acc, [ix + VP], cy, mask=valid)
                plsc.addupdate_scatter(acc, [ix + 2 * VP], cz, mask=valid)
            return carry
        lax.fori_loop(0, FPW // 16, fbody, 0)
        pltpu.sync_copy(acc, out_hbm.at[t, wid])

    for b in range(B):
        run_table(mesh_hbm, b, fidxm, b)
        pltpu.sync_copy(rface_hbm.at[b, pl.ds(wid * FPW * 3, FPW * 3)], fidxr)
        run_table(recon_hbm, b, fidxr, B + b)


def _normals_partials(mesh2, recon2, facep, rfacep):
    B = mesh2.shape[0]
    run = pl.kernel(
        _normals_body,
        out_type=jax.ShapeDtypeStruct((2 * B, NW, 3 * VP), jnp.float32),
        mesh=_SC_MESH,
        compiler_params=_SC_PARAMS,
        scratch_types=[
            pltpu.VMEM((3 * V,), jnp.float32),
            pltpu.VMEM((3 * FPW,), jnp.int32),
            pltpu.VMEM((3 * FPW,), jnp.int32),
            pltpu.VMEM((3 * VP,), jnp.float32),
            pltpu.SemaphoreType.DMA,
        ],
    )
    return run(mesh2, recon2, facep, rfacep)


NR = TI // 128


def _knn_body(q_ref, tT_ref, rowd_ref, rowi_ref, cold_ref, coli_ref):
    i = pl.program_id(1)
    q = q_ref[...]
    tT = tT_ref[0]
    dx = q[:, 0:1] - tT[0:1, :]
    dy = q[:, 1:2] - tT[1:2, :]
    dz = q[:, 2:3] - tT[2:3, :]
    d2 = dx * dx + dy * dy + dz * dz
    iota_j = lax.broadcasted_iota(jnp.int32, (TI, VP), 1)
    iota_i = lax.broadcasted_iota(jnp.int32, (TI, VP), 0) + i * TI
    rmin = jnp.min(d2, axis=1)
    rarg = jnp.min(jnp.where(d2 == rmin[:, None], iota_j, 2**30), axis=1)
    rowd_ref[0, pl.ds(i * NR, NR)] = rmin.reshape(NR, 128)
    rowi_ref[0, pl.ds(i * NR, NR)] = rarg.reshape(NR, 128)
    cmin = jnp.min(d2, axis=0)
    carg = jnp.min(jnp.where(d2 == cmin[None, :], iota_i, 2**30), axis=0)

    @pl.when(i == 0)
    def _():
        cold_ref[0] = cmin.reshape(VP // 128, 128)
        coli_ref[0] = carg.reshape(VP // 128, 128)

    @pl.when(i > 0)
    def _():
        cprev = cold_ref[0].reshape(VP)
        upd = cmin < cprev
        cold_ref[0] = jnp.where(upd, cmin, cprev).reshape(VP // 128, 128)
        coli_ref[0] = jnp.where(
            upd, carg, coli_ref[0].reshape(VP)).reshape(VP // 128, 128)


def _knn_both(meshes_p, recons_pT):
    B = meshes_p.shape[0]
    ni = VP // TI
    grid = (B, ni)
    outs = pl.pallas_call(
        _knn_body,
        grid=grid,
        in_specs=[
            pl.BlockSpec((None, TI, 3), lambda b, i: (b, i, 0)),
            pl.BlockSpec((1, 3, VP), lambda b, i: (b, 0, 0)),
        ],
        out_specs=[
            pl.BlockSpec((1, VP // 128, 128), lambda b, i: (b, 0, 0)),
            pl.BlockSpec((1, VP // 128, 128), lambda b, i: (b, 0, 0)),
            pl.BlockSpec((1, VP // 128, 128), lambda b, i: (b, 0, 0)),
            pl.BlockSpec((1, VP // 128, 128), lambda b, i: (b, 0, 0)),
        ],
        out_shape=[
            jax.ShapeDtypeStruct((B, VP // 128, 128), jnp.float32),
            jax.ShapeDtypeStruct((B, VP // 128, 128), jnp.int32),
            jax.ShapeDtypeStruct((B, VP // 128, 128), jnp.float32),
            jax.ShapeDtypeStruct((B, VP // 128, 128), jnp.int32),
        ],
    )(meshes_p, recons_pT)
    rowd, rowi, cold, coli = outs
    return (rowd.reshape(B, VP), rowi.reshape(B, VP),
            cold.reshape(B, VP), coli.reshape(B, VP))


def _norm_body(p_ref, n_ref):
    vn = jnp.sum(p_ref[0], axis=0)
    norm = jnp.sqrt(jnp.sum(vn * vn, axis=0, keepdims=True))
    n_ref[0] = vn / jnp.maximum(norm, 1e-12)


def _normalize(partials):
    nt = partials.shape[0]
    return pl.pallas_call(
        _norm_body,
        grid=(nt,),
        in_specs=[pl.BlockSpec((1, NW, 3, VP), lambda t: (t, 0, 0, 0))],
        out_specs=pl.BlockSpec((1, 3, VP), lambda t: (t, 0, 0)),
        out_shape=jax.ShapeDtypeStruct((nt, 3, VP), jnp.float32),
    )(partials)


def _loss_body(mesh_hbm, recon_hbm, mn_hbm, rn_hbm,
               ir_hbm, dr_hbm, ic_hbm, dc_hbm, out_hbm,
               mtab, rtab, rntab, mnbuf, irbuf, drbuf, icbuf, dcbuf, outbuf,
               sem):
    cid = lax.axis_index("c")
    sid = lax.axis_index("s")
    wid = sid * 2 + cid
    B = mesh_hbm.shape[0]
    lane = lax.iota(jnp.int32, 16)
    zeros = jnp.zeros((16,), jnp.float32)
    for b in range(B):
        cps = [
            pltpu.async_copy(mesh_hbm.at[b], mtab, sem),
            pltpu.async_copy(recon_hbm.at[b], rtab, sem),
            pltpu.async_copy(rn_hbm.at[b], rntab, sem),
            pltpu.async_copy(mn_hbm.at[b, :, pl.ds(wid * QPW, QPW)], mnbuf,
                             sem),
            pltpu.async_copy(ir_hbm.at[b, pl.ds(wid * QPW, QPW)], irbuf, sem),
            pltpu.async_copy(dr_hbm.at[b, pl.ds(wid * QPW, QPW)], drbuf, sem),
            pltpu.async_copy(ic_hbm.at[b, pl.ds(wid * QPW, QPW)], icbuf, sem),
            pltpu.async_copy(dc_hbm.at[b, pl.ds(wid * QPW, QPW)], dcbuf, sem),
        ]
        for cp in cps:
            cp.wait()

        def kbody(k, carry):
            a0, a1, a2 = carry
            base = k * 16
            gbase = wid * QPW + base
            valid = (gbase + lane) < V
            idx = irbuf[pl.ds(base, 16)]
            d2r = drbuf[pl.ds(base, 16)]
            nnx = plsc.load_gather(rtab, [idx])
            nny = plsc.load_gather(rtab, [idx + VP])
            nnz = plsc.load_gather(rtab, [idx + 2 * VP])
            rnx = plsc.load_gather(rntab, [idx])
            rny = plsc.load_gather(rntab, [idx + VP])
            rnz = plsc.load_gather(rntab, [idx + 2 * VP])
            mx = mtab[pl.ds(gbase, 16)]
            my = mtab[pl.ds(VP + gbase, 16)]
            mz = mtab[pl.ds(2 * VP + gbase, 16)]
            mnx = mnbuf[0, pl.ds(base, 16)]
            mny = mnbuf[1, pl.ds(base, 16)]
            mnz = mnbuf[2, pl.ds(base, 16)]
            ndot = mnx * rnx + mny * rny + mnz * rnz
            ok1 = valid & (ndot >= 0.45) & (d2r <= D2_THRESH)
            w1 = jnp.where(ok1, 1.0, 0.0)
            a0 = a0 + jnp.abs((mx - nnx) * rnx) * w1
            a1 = a1 + jnp.abs((my - nny) * rny) * w1
            a2 = a2 + jnp.abs((mz - nnz) * rnz) * w1
            idc = icbuf[pl.ds(base, 16)]
            d2c = dcbuf[pl.ds(base, 16)]
            gx = plsc.load_gather(mtab, [idc])
            gy = plsc.load_gather(mtab, [idc + VP])
            gz = plsc.load_gather(mtab, [idc + 2 * VP])
            rx = rtab[pl.ds(gbase, 16)]
            ry = rtab[pl.ds(VP + gbase, 16)]
            rz = rtab[pl.ds(2 * VP + gbase, 16)]
            w2 = jnp.where(valid & (d2c <= D2_THRESH), 1.0, 0.0)
            a0 = a0 + jnp.abs(rx - gx) * w2
            a1 = a1 + jnp.abs(ry - gy) * w2
            a2 = a2 + jnp.abs(rz - gz) * w2
            return (a0, a1, a2)

        a0, a1, a2 = lax.fori_loop(0, QPW // 16, kbody, (zeros, zeros, zeros))
        outbuf[0] = a0
        outbuf[1] = a1
        outbuf[2] = a2
        outbuf[3] = zeros
        pltpu.sync_copy(outbuf, out_hbm.at[b, wid])


def _loss_partials(mesh_t, recon_t, mesh_nT, rec_nT, ir, dr, ic, dc):
    B = mesh_t.shape[0]
    run = pl.kernel(
        _loss_body,
        out_type=jax.ShapeDtypeStruct((B, NW, 4, 16), jnp.float32),
        mesh=_SC_MESH,
        compiler_params=_SC_PARAMS,
        scratch_types=[
            pltpu.VMEM((3 * VP,), jnp.float32),
            pltpu.VMEM((3 * VP,), jnp.float32),
            pltpu.VMEM((3 * VP,), jnp.float32),
            pltpu.VMEM((3, QPW), jnp.float32),
            pltpu.VMEM((QPW,), jnp.int32),
            pltpu.VMEM((QPW,), jnp.float32),
            pltpu.VMEM((QPW,), jnp.int32),
            pltpu.VMEM((QPW,), jnp.float32),
            pltpu.VMEM((4, 16), jnp.float32),
            pltpu.SemaphoreType.DMA,
        ],
    )
    return run(mesh_t.reshape(B, 3 * VP), recon_t.reshape(B, 3 * VP),
               mesh_nT, rec_nT.reshape(B, 3 * VP), ir, dr, ic, dc)


def _fin_body(p_ref, o_ref):
    x = p_ref[0]
    s = jnp.sum(jnp.sum(x, axis=0), axis=1)
    o_ref[0] = s * (1.0 / V)


def _finalize(psums):
    B = psums.shape[0]
    out = pl.pallas_call(
        _fin_body,
        grid=(B,),
        in_specs=[pl.BlockSpec((1, NW, 4, 16), lambda b: (b, 0, 0, 0))],
        out_specs=pl.BlockSpec((None, 1, 4), lambda b: (b, 0, 0)),
        out_shape=jax.ShapeDtypeStruct((B, 1, 4), jnp.float32),
    )(psums)
    return out[:, 0, :3]


@jax.jit
def _impl(joints, meshes, recons, face, recons_faces):
    B = meshes.shape[0]
    pad = ((0, 0), (0, VP - V), (0, 0))
    meshes_p = jnp.pad(meshes, pad, constant_values=PAD_COORD)
    recons_p = jnp.pad(recons, pad, constant_values=PAD_COORD)
    meshes_pT = jnp.transpose(meshes_p, (0, 2, 1))
    recons_pT = jnp.transpose(recons_p, (0, 2, 1))

    facep = jnp.pad(face, ((0, FP - F), (0, 0))).reshape(3 * FP)
    rfacep = jnp.pad(recons_faces,
                     ((0, 0), (0, FP - F), (0, 0))).reshape(B, 3 * FP)
    partials = _normals_partials(
        meshes.reshape(B, 3 * V), recons.reshape(B, 3 * V), facep, rfacep)
    normals = _normalize(partials.reshape(2 * B, NW, 3, VP))
    mesh_nT, rec_nT = normals[:B], normals[B:]

    row_d2, row_idx, col_d2, col_idx = _knn_both(meshes_p, recons_pT)

    psums = _loss_partials(meshes_pT, recons_pT, mesh_nT, rec_nT,
                           row_idx, row_d2, col_idx, col_d2)
    return _finalize(psums)


def kernel(joints, meshes, recons, face, recons_faces):
    return _impl(joints, meshes, recons, face, recons_faces)


# speedup vs baseline: 1.1442x; 1.0109x over previous
import functools

import jax
import jax.numpy as jnp
from jax import lax
from jax.experimental import pallas as pl
from jax.experimental.pallas import tpu as pltpu
from jax.experimental.pallas import tpu_sc as plsc

V = 5000
VP = 5120
F = 10000
PAD_COORD = 1.0e6
TI = 1024
NW = 32
FPW = 320
FP = NW * FPW
QPW = VP // NW
D2_THRESH = 0.005 * 0.005

_SC_MESH = plsc.VectorSubcoreMesh(core_axis_name="c", subcore_axis_name="s")
_SC_PARAMS = pltpu.CompilerParams(
    use_tc_tiling_on_sc=False, needs_layout_passes=False)


def _normals_body(mesh_hbm, recon_hbm, face_hbm, rface_hbm, out_hbm,
                  vtab, fidxm, fidxr, acc, sem):
    cid = lax.axis_index("c")
    sid = lax.axis_index("s")
    wid = sid * 2 + cid
    B = mesh_hbm.shape[0]
    lane = lax.iota(jnp.int32, 16)
    pltpu.sync_copy(face_hbm.at[pl.ds(wid * FPW * 3, FPW * 3)], fidxm)

    def run_table(verts_src, b, fidx, t):
        cp = pltpu.async_copy(verts_src.at[b], vtab, sem)

        def zbody(k, carry):
            acc[pl.ds(k * 16, 16)] = jnp.zeros((16,), jnp.float32)
            return carry
        lax.fori_loop(0, (3 * VP) // 16, zbody, 0)
        cp.wait()

        def fbody(k, carry):
            base = k * 16
            valid = (wid * FPW + base + lane) < F
            idx3 = (base + lane) * 3
            i0 = plsc.load_gather(fidx, [idx3])
            i1 = plsc.load_gather(fidx, [idx3 + 1])
            i2 = plsc.load_gather(fidx, [idx3 + 2])

            def g(ix, c):
                return plsc.load_gather(vtab, [ix * 3 + c])
            v0x, v0y, v0z = g(i0, 0), g(i0, 1), g(i0, 2)
            e1x = g(i1, 0) - v0x
            e1y = g(i1, 1) - v0y
            e1z = g(i1, 2) - v0z
            e2x = g(i2, 0) - v0x
            e2y = g(i2, 1) - v0y
            e2z = g(i2, 2) - v0z
            cx = e1y * e2z - e1z * e2y
            cy = e1z * e2x - e1x * e2z
            cz = e1x * e2y - e1y * e2x
            for ix in (i0, i1, i2):
                plsc.addupdate_scatter(acc, [ix], cx, mask=valid)
                plsc.addupdate_scatter(acc, [ix + VP], cy, mask=valid)
                plsc.addupdate_scatter(acc, [ix + 2 * VP], cz, mask=valid)
            return carry
        lax.fori_loop(0, FPW // 16, fbody, 0)
        pltpu.sync_copy(acc, out_hbm.at[t, wid])

    for b in range(B):
        run_table(mesh_hbm, b, fidxm, b)
        pltpu.sync_copy(rface_hbm.at[b, pl.ds(wid * FPW * 3, FPW * 3)], fidxr)
        run_table(recon_hbm, b, fidxr, B + b)


def _normals_partials(mesh2, recon2, facep, rfacep):
    B = mesh2.shape[0]
    run = pl.kernel(
        _normals_body,
        out_type=jax.ShapeDtypeStruct((2 * B, NW, 3 * VP), jnp.float32),
        mesh=_SC_MESH,
        compiler_params=_SC_PARAMS,
        scratch_types=[
            pltpu.VMEM((3 * V,), jnp.float32),
            pltpu.VMEM((3 * FPW,), jnp.int32),
            pltpu.VMEM((3 * FPW,), jnp.int32),
            pltpu.VMEM((3 * VP,), jnp.float32),
            pltpu.SemaphoreType.DMA,
        ],
    )
    return run(mesh2, recon2, facep, rfacep)


NR = TI // 128


def _knn_body(q_ref, tT_ref, rowd_ref, rowi_ref, cold_ref, coli_ref):
    i = pl.program_id(1)
    q = q_ref[...]
    tT = tT_ref[0]
    dx = q[:, 0:1] - tT[0:1, :]
    dy = q[:, 1:2] - tT[1:2, :]
    dz = q[:, 2:3] - tT[2:3, :]
    d2 = dx * dx + dy * dy + dz * dz
    iota_j = lax.broadcasted_iota(jnp.int32, (TI, VP), 1)
    iota_i = lax.broadcasted_iota(jnp.int32, (TI, VP), 0) + i * TI
    rmin = jnp.min(d2, axis=1)
    rarg = jnp.min(jnp.where(d2 == rmin[:, None], iota_j, 2**30), axis=1)
    rowd_ref[0, pl.ds(i * NR, NR)] = rmin.reshape(NR, 128)
    rowi_ref[0, pl.ds(i * NR, NR)] = rarg.reshape(NR, 128)
    cmin = jnp.min(d2, axis=0)
    carg = jnp.min(jnp.where(d2 == cmin[None, :], iota_i, 2**30), axis=0)

    @pl.when(i == 0)
    def _():
        cold_ref[0] = cmin.reshape(VP // 128, 128)
        coli_ref[0] = carg.reshape(VP // 128, 128)

    @pl.when(i > 0)
    def _():
        cprev = cold_ref[0].reshape(VP)
        upd = cmin < cprev
        cold_ref[0] = jnp.where(upd, cmin, cprev).reshape(VP // 128, 128)
        coli_ref[0] = jnp.where(
            upd, carg, coli_ref[0].reshape(VP)).reshape(VP // 128, 128)


def _knn_both(meshes_p, recons_pT):
    B = meshes_p.shape[0]
    ni = VP // TI
    grid = (B, ni)
    outs = pl.pallas_call(
        _knn_body,
        grid=grid,
        compiler_params=pltpu.CompilerParams(
            vmem_limit_bytes=100 * 1024 * 1024),
        in_specs=[
            pl.BlockSpec((None, TI, 3), lambda b, i: (b, i, 0)),
            pl.BlockSpec((1, 3, VP), lambda b, i: (b, 0, 0)),
        ],
        out_specs=[
            pl.BlockSpec((1, VP // 128, 128), lambda b, i: (b, 0, 0)),
            pl.BlockSpec((1, VP // 128, 128), lambda b, i: (b, 0, 0)),
            pl.BlockSpec((1, VP // 128, 128), lambda b, i: (b, 0, 0)),
            pl.BlockSpec((1, VP // 128, 128), lambda b, i: (b, 0, 0)),
        ],
        out_shape=[
            jax.ShapeDtypeStruct((B, VP // 128, 128), jnp.float32),
            jax.ShapeDtypeStruct((B, VP // 128, 128), jnp.int32),
            jax.ShapeDtypeStruct((B, VP // 128, 128), jnp.float32),
            jax.ShapeDtypeStruct((B, VP // 128, 128), jnp.int32),
        ],
    )(meshes_p, recons_pT)
    rowd, rowi, cold, coli = outs
    return (rowd.reshape(B, VP), rowi.reshape(B, VP),
            cold.reshape(B, VP), coli.reshape(B, VP))


def _norm_body(p_ref, n_ref):
    vn = jnp.sum(p_ref[0], axis=0)
    norm = jnp.sqrt(jnp.sum(vn * vn, axis=0, keepdims=True))
    n_ref[0] = vn / jnp.maximum(norm, 1e-12)


def _normalize(partials):
    nt = partials.shape[0]
    return pl.pallas_call(
        _norm_body,
        grid=(nt,),
        in_specs=[pl.BlockSpec((1, NW, 3, VP), lambda t: (t, 0, 0, 0))],
        out_specs=pl.BlockSpec((1, 3, VP), lambda t: (t, 0, 0)),
        out_shape=jax.ShapeDtypeStruct((nt, 3, VP), jnp.float32),
    )(partials)


def _loss_body(mesh_hbm, recon_hbm, mn_hbm, rn_hbm,
               ir_hbm, dr_hbm, ic_hbm, dc_hbm, out_hbm,
               mtab, rtab, rntab, mnbuf, irbuf, drbuf, icbuf, dcbuf, outbuf,
               sem):
    cid = lax.axis_index("c")
    sid = lax.axis_index("s")
    wid = sid * 2 + cid
    B = mesh_hbm.shape[0]
    lane = lax.iota(jnp.int32, 16)
    zeros = jnp.zeros((16,), jnp.float32)
    for b in range(B):
        cps = [
            pltpu.async_copy(mesh_hbm.at[b], mtab, sem),
            pltpu.async_copy(recon_hbm.at[b], rtab, sem),
            pltpu.async_copy(rn_hbm.at[b], rntab, sem),
            pltpu.async_copy(mn_hbm.at[b, :, pl.ds(wid * QPW, QPW)], mnbuf,
                             sem),
            pltpu.async_copy(ir_hbm.at[b, pl.ds(wid * QPW, QPW)], irbuf, sem),
            pltpu.async_copy(dr_hbm.at[b, pl.ds(wid * QPW, QPW)], drbuf, sem),
            pltpu.async_copy(ic_hbm.at[b, pl.ds(wid * QPW, QPW)], icbuf, sem),
            pltpu.async_copy(dc_hbm.at[b, pl.ds(wid * QPW, QPW)], dcbuf, sem),
        ]
        for cp in cps:
            cp.wait()

        def kbody(k, carry):
            a0, a1, a2 = carry
            base = k * 16
            gbase = wid * QPW + base
            valid = (gbase + lane) < V
            idx = irbuf[pl.ds(base, 16)]
            d2r = drbuf[pl.ds(base, 16)]
            nnx = plsc.load_gather(rtab, [idx])
            nny = plsc.load_gather(rtab, [idx + VP])
            nnz = plsc.load_gather(rtab, [idx + 2 * VP])
            rnx = plsc.load_gather(rntab, [idx])
            rny = plsc.load_gather(rntab, [idx + VP])
            rnz = plsc.load_gather(rntab, [idx + 2 * VP])
            mx = mtab[pl.ds(gbase, 16)]
            my = mtab[pl.ds(VP + gbase, 16)]
            mz = mtab[pl.ds(2 * VP + gbase, 16)]
            mnx = mnbuf[0, pl.ds(base, 16)]
            mny = mnbuf[1, pl.ds(base, 16)]
            mnz = mnbuf[2, pl.ds(base, 16)]
            ndot = mnx * rnx + mny * rny + mnz * rnz
            ok1 = valid & (ndot >= 0.45) & (d2r <= D2_THRESH)
            w1 = jnp.where(ok1, 1.0, 0.0)
            a0 = a0 + jnp.abs((mx - nnx) * rnx) * w1
            a1 = a1 + jnp.abs((my - nny) * rny) * w1
            a2 = a2 + jnp.abs((mz - nnz) * rnz) * w1
            idc = icbuf[pl.ds(base, 16)]
            d2c = dcbuf[pl.ds(base, 16)]
            gx = plsc.load_gather(mtab, [idc])
            gy = plsc.load_gather(mtab, [idc + VP])
            gz = plsc.load_gather(mtab, [idc + 2 * VP])
            rx = rtab[pl.ds(gbase, 16)]
            ry = rtab[pl.ds(VP + gbase, 16)]
            rz = rtab[pl.ds(2 * VP + gbase, 16)]
            w2 = jnp.where(valid & (d2c <= D2_THRESH), 1.0, 0.0)
            a0 = a0 + jnp.abs(rx - gx) * w2
            a1 = a1 + jnp.abs(ry - gy) * w2
            a2 = a2 + jnp.abs(rz - gz) * w2
            return (a0, a1, a2)

        a0, a1, a2 = lax.fori_loop(0, QPW // 16, kbody, (zeros, zeros, zeros))
        outbuf[0] = a0
        outbuf[1] = a1
        outbuf[2] = a2
        outbuf[3] = zeros
        pltpu.sync_copy(outbuf, out_hbm.at[b, wid])


def _loss_partials(mesh_t, recon_t, mesh_nT, rec_nT, ir, dr, ic, dc):
    B = mesh_t.shape[0]
    run = pl.kernel(
        _loss_body,
        out_type=jax.ShapeDtypeStruct((B, NW, 4, 16), jnp.float32),
        mesh=_SC_MESH,
        compiler_params=_SC_PARAMS,
        scratch_types=[
            pltpu.VMEM((3 * VP,), jnp.float32),
            pltpu.VMEM((3 * VP,), jnp.float32),
            pltpu.VMEM((3 * VP,), jnp.float32),
            pltpu.VMEM((3, QPW), jnp.float32),
            pltpu.VMEM((QPW,), jnp.int32),
            pltpu.VMEM((QPW,), jnp.float32),
            pltpu.VMEM((QPW,), jnp.int32),
            pltpu.VMEM((QPW,), jnp.float32),
            pltpu.VMEM((4, 16), jnp.float32),
            pltpu.SemaphoreType.DMA,
        ],
    )
    return run(mesh_t.reshape(B, 3 * VP), recon_t.reshape(B, 3 * VP),
               mesh_nT, rec_nT.reshape(B, 3 * VP), ir, dr, ic, dc)


def _fin_body(p_ref, o_ref):
    x = p_ref[0]
    s = jnp.sum(jnp.sum(x, axis=0), axis=1)
    o_ref[0] = s * (1.0 / V)


def _finalize(psums):
    B = psums.shape[0]
    out = pl.pallas_call(
        _fin_body,
        grid=(B,),
        in_specs=[pl.BlockSpec((1, NW, 4, 16), lambda b: (b, 0, 0, 0))],
        out_specs=pl.BlockSpec((None, 1, 4), lambda b: (b, 0, 0)),
        out_shape=jax.ShapeDtypeStruct((B, 1, 4), jnp.float32),
    )(psums)
    return out[:, 0, :3]


@jax.jit
def _impl(joints, meshes, recons, face, recons_faces):
    B = meshes.shape[0]
    pad = ((0, 0), (0, VP - V), (0, 0))
    meshes_p = jnp.pad(meshes, pad, constant_values=PAD_COORD)
    recons_p = jnp.pad(recons, pad, constant_values=PAD_COORD)
    meshes_pT = jnp.transpose(meshes_p, (0, 2, 1))
    recons_pT = jnp.transpose(recons_p, (0, 2, 1))

    facep = jnp.pad(face, ((0, FP - F), (0, 0))).reshape(3 * FP)
    rfacep = jnp.pad(recons_faces,
                     ((0, 0), (0, FP - F), (0, 0))).reshape(B, 3 * FP)
    partials = _normals_partials(
        meshes.reshape(B, 3 * V), recons.reshape(B, 3 * V), facep, rfacep)
    normals = _normalize(partials.reshape(2 * B, NW, 3, VP))
    mesh_nT, rec_nT = normals[:B], normals[B:]

    row_d2, row_idx, col_d2, col_idx = _knn_both(meshes_p, recons_pT)

    psums = _loss_partials(meshes_pT, recons_pT, mesh_nT, rec_nT,
                           row_idx, row_d2, col_idx, col_d2)
    return _finalize(psums)


def kernel(joints, meshes, recons, face, recons_faces):
    return _impl(joints, meshes, recons, face, recons_faces)


# speedup vs baseline: 1.1454x; 1.0010x over previous
import functools

import jax
import jax.numpy as jnp
from jax import lax
from jax.experimental import pallas as pl
from jax.experimental.pallas import tpu as pltpu
from jax.experimental.pallas import tpu_sc as plsc

V = 5000
VP = 5120
F = 10000
PAD_COORD = 1.0e6
TI = 1280
NW = 32
FPW = 320
FP = NW * FPW
QPW = VP // NW
D2_THRESH = 0.005 * 0.005

_SC_MESH = plsc.VectorSubcoreMesh(core_axis_name="c", subcore_axis_name="s")
_SC_PARAMS = pltpu.CompilerParams(
    use_tc_tiling_on_sc=False, needs_layout_passes=False)


def _normals_body(mesh_hbm, recon_hbm, face_hbm, rface_hbm, out_hbm,
                  vtab, fidxm, fidxr, acc, sem):
    cid = lax.axis_index("c")
    sid = lax.axis_index("s")
    wid = sid * 2 + cid
    B = mesh_hbm.shape[0]
    lane = lax.iota(jnp.int32, 16)
    pltpu.sync_copy(face_hbm.at[pl.ds(wid * FPW * 3, FPW * 3)], fidxm)

    def run_table(verts_src, b, fidx, t):
        cp = pltpu.async_copy(verts_src.at[b], vtab, sem)

        def zbody(k, carry):
            acc[pl.ds(k * 16, 16)] = jnp.zeros((16,), jnp.float32)
            return carry
        lax.fori_loop(0, (3 * VP) // 16, zbody, 0)
        cp.wait()

        def fbody(k, carry):
            base = k * 16
            valid = (wid * FPW + base + lane) < F
            idx3 = (base + lane) * 3
            i0 = plsc.load_gather(fidx, [idx3])
            i1 = plsc.load_gather(fidx, [idx3 + 1])
            i2 = plsc.load_gather(fidx, [idx3 + 2])

            def g(ix, c):
                return plsc.load_gather(vtab, [ix * 3 + c])
            v0x, v0y, v0z = g(i0, 0), g(i0, 1), g(i0, 2)
            e1x = g(i1, 0) - v0x
            e1y = g(i1, 1) - v0y
            e1z = g(i1, 2) - v0z
            e2x = g(i2, 0) - v0x
            e2y = g(i2, 1) - v0y
            e2z = g(i2, 2) - v0z
            cx = e1y * e2z - e1z * e2y
            cy = e1z * e2x - e1x * e2z
            cz = e1x * e2y - e1y * e2x
            for ix in (i0, i1, i2):
                plsc.addupdate_scatter(acc, [ix], cx, mask=valid)
                plsc.addupdate_scatter(acc, [ix + VP], cy, mask=valid)
                plsc.addupdate_scatter(acc, [ix + 2 * VP], cz, mask=valid)
            return carry
        lax.fori_loop(0, FPW // 16, fbody, 0)
        pltpu.sync_copy(acc, out_hbm.at[t, wid])

    for b in range(B):
        run_table(mesh_hbm, b, fidxm, b)
        pltpu.sync_copy(rface_hbm.at[b, pl.ds(wid * FPW * 3, FPW * 3)], fidxr)
        run_table(recon_hbm, b, fidxr, B + b)


def _normals_partials(mesh2, recon2, facep, rfacep):
    B = mesh2.shape[0]
    run = pl.kernel(
        _normals_body,
        out_type=jax.ShapeDtypeStruct((2 * B, NW, 3 * VP), jnp.float32),
        mesh=_SC_MESH,
        compiler_params=_SC_PARAMS,
        scratch_types=[
            pltpu.VMEM((3 * V,), jnp.float32),
            pltpu.VMEM((3 * FPW,), jnp.int32),
            pltpu.VMEM((3 * FPW,), jnp.int32),
            pltpu.VMEM((3 * VP,), jnp.float32),
            pltpu.SemaphoreType.DMA,
        ],
    )
    return run(mesh2, recon2, facep, rfacep)


NR = TI // 128


def _knn_body(q_ref, tT_ref, rowd_ref, rowi_ref, cold_ref, coli_ref):
    i = pl.program_id(1)
    q = q_ref[...]
    tT = tT_ref[0]
    dx = q[:, 0:1] - tT[0:1, :]
    dy = q[:, 1:2] - tT[1:2, :]
    dz = q[:, 2:3] - tT[2:3, :]
    d2 = dx * dx + dy * dy + dz * dz
    iota_j = lax.broadcasted_iota(jnp.int32, (TI, VP), 1)
    iota_i = lax.broadcasted_iota(jnp.int32, (TI, VP), 0) + i * TI
    rmin = jnp.min(d2, axis=1)
    rarg = jnp.min(jnp.where(d2 == rmin[:, None], iota_j, 2**30), axis=1)
    rowd_ref[0, pl.ds(i * NR, NR)] = rmin.reshape(NR, 128)
    rowi_ref[0, pl.ds(i * NR, NR)] = rarg.reshape(NR, 128)
    cmin = jnp.min(d2, axis=0)
    carg = jnp.min(jnp.where(d2 == cmin[None, :], iota_i, 2**30), axis=0)

    @pl.when(i == 0)
    def _():
        cold_ref[0] = cmin.reshape(VP // 128, 128)
        coli_ref[0] = carg.reshape(VP // 128, 128)

    @pl.when(i > 0)
    def _():
        cprev = cold_ref[0].reshape(VP)
        upd = cmin < cprev
        cold_ref[0] = jnp.where(upd, cmin, cprev).reshape(VP // 128, 128)
        coli_ref[0] = jnp.where(
            upd, carg, coli_ref[0].reshape(VP)).reshape(VP // 128, 128)


def _knn_both(meshes_p, recons_pT):
    B = meshes_p.shape[0]
    ni = VP // TI
    grid = (B, ni)
    outs = pl.pallas_call(
        _knn_body,
        grid=grid,
        compiler_params=pltpu.CompilerParams(
            vmem_limit_bytes=100 * 1024 * 1024),
        in_specs=[
            pl.BlockSpec((None, TI, 3), lambda b, i: (b, i, 0)),
            pl.BlockSpec((1, 3, VP), lambda b, i: (b, 0, 0)),
        ],
        out_specs=[
            pl.BlockSpec((1, VP // 128, 128), lambda b, i: (b, 0, 0)),
            pl.BlockSpec((1, VP // 128, 128), lambda b, i: (b, 0, 0)),
            pl.BlockSpec((1, VP // 128, 128), lambda b, i: (b, 0, 0)),
            pl.BlockSpec((1, VP // 128, 128), lambda b, i: (b, 0, 0)),
        ],
        out_shape=[
            jax.ShapeDtypeStruct((B, VP // 128, 128), jnp.float32),
            jax.ShapeDtypeStruct((B, VP // 128, 128), jnp.int32),
            jax.ShapeDtypeStruct((B, VP // 128, 128), jnp.float32),
            jax.ShapeDtypeStruct((B, VP // 128, 128), jnp.int32),
        ],
    )(meshes_p, recons_pT)
    rowd, rowi, cold, coli = outs
    return (rowd.reshape(B, VP), rowi.reshape(B, VP),
            cold.reshape(B, VP), coli.reshape(B, VP))


def _norm_body(p_ref, n_ref):
    vn = jnp.sum(p_ref[0], axis=0)
    norm = jnp.sqrt(jnp.sum(vn * vn, axis=0, keepdims=True))
    n_ref[0] = vn / jnp.maximum(norm, 1e-12)


def _normalize(partials):
    nt = partials.shape[0]
    return pl.pallas_call(
        _norm_body,
        grid=(nt,),
        in_specs=[pl.BlockSpec((1, NW, 3, VP), lambda t: (t, 0, 0, 0))],
        out_specs=pl.BlockSpec((1, 3, VP), lambda t: (t, 0, 0)),
        out_shape=jax.ShapeDtypeStruct((nt, 3, VP), jnp.float32),
    )(partials)


def _loss_body(mesh_hbm, recon_hbm, mn_hbm, rn_hbm,
               ir_hbm, dr_hbm, ic_hbm, dc_hbm, out_hbm,
               mtab, rtab, rntab, mnbuf, irbuf, drbuf, icbuf, dcbuf, outbuf,
               sem):
    cid = lax.axis_index("c")
    sid = lax.axis_index("s")
    wid = sid * 2 + cid
    B = mesh_hbm.shape[0]
    lane = lax.iota(jnp.int32, 16)
    zeros = jnp.zeros((16,), jnp.float32)
    for b in range(B):
        cps = [
            pltpu.async_copy(mesh_hbm.at[b], mtab, sem),
            pltpu.async_copy(recon_hbm.at[b], rtab, sem),
            pltpu.async_copy(rn_hbm.at[b], rntab, sem),
            pltpu.async_copy(mn_hbm.at[b, :, pl.ds(wid * QPW, QPW)], mnbuf,
                             sem),
            pltpu.async_copy(ir_hbm.at[b, pl.ds(wid * QPW, QPW)], irbuf, sem),
            pltpu.async_copy(dr_hbm.at[b, pl.ds(wid * QPW, QPW)], drbuf, sem),
            pltpu.async_copy(ic_hbm.at[b, pl.ds(wid * QPW, QPW)], icbuf, sem),
            pltpu.async_copy(dc_hbm.at[b, pl.ds(wid * QPW, QPW)], dcbuf, sem),
        ]
        for cp in cps:
            cp.wait()

        def kbody(k, carry):
            a0, a1, a2 = carry
            base = k * 16
            gbase = wid * QPW + base
            valid = (gbase + lane) < V
            idx = irbuf[pl.ds(base, 16)]
            d2r = drbuf[pl.ds(base, 16)]
            nnx = plsc.load_gather(rtab, [idx])
            nny = plsc.load_gather(rtab, [idx + VP])
            nnz = plsc.load_gather(rtab, [idx + 2 * VP])
            rnx = plsc.load_gather(rntab, [idx])
            rny = plsc.load_gather(rntab, [idx + VP])
            rnz = plsc.load_gather(rntab, [idx + 2 * VP])
            mx = mtab[pl.ds(gbase, 16)]
            my = mtab[pl.ds(VP + gbase, 16)]
            mz = mtab[pl.ds(2 * VP + gbase, 16)]
            mnx = mnbuf[0, pl.ds(base, 16)]
            mny = mnbuf[1, pl.ds(base, 16)]
            mnz = mnbuf[2, pl.ds(base, 16)]
            ndot = mnx * rnx + mny * rny + mnz * rnz
            ok1 = valid & (ndot >= 0.45) & (d2r <= D2_THRESH)
            w1 = jnp.where(ok1, 1.0, 0.0)
            a0 = a0 + jnp.abs((mx - nnx) * rnx) * w1
            a1 = a1 + jnp.abs((my - nny) * rny) * w1
            a2 = a2 + jnp.abs((mz - nnz) * rnz) * w1
            idc = icbuf[pl.ds(base, 16)]
            d2c = dcbuf[pl.ds(base, 16)]
            gx = plsc.load_gather(mtab, [idc])
            gy = plsc.load_gather(mtab, [idc + VP])
            gz = plsc.load_gather(mtab, [idc + 2 * VP])
            rx = rtab[pl.ds(gbase, 16)]
            ry = rtab[pl.ds(VP + gbase, 16)]
            rz = rtab[pl.ds(2 * VP + gbase, 16)]
            w2 = jnp.where(valid & (d2c <= D2_THRESH), 1.0, 0.0)
            a0 = a0 + jnp.abs(rx - gx) * w2
            a1 = a1 + jnp.abs(ry - gy) * w2
            a2 = a2 + jnp.abs(rz - gz) * w2
            return (a0, a1, a2)

        a0, a1, a2 = lax.fori_loop(0, QPW // 16, kbody, (zeros, zeros, zeros))
        outbuf[0] = a0
        outbuf[1] = a1
        outbuf[2] = a2
        outbuf[3] = zeros
        pltpu.sync_copy(outbuf, out_hbm.at[b, wid])


def _loss_partials(mesh_t, recon_t, mesh_nT, rec_nT, ir, dr, ic, dc):
    B = mesh_t.shape[0]
    run = pl.kernel(
        _loss_body,
        out_type=jax.ShapeDtypeStruct((B, NW, 4, 16), jnp.float32),
        mesh=_SC_MESH,
        compiler_params=_SC_PARAMS,
        scratch_types=[
            pltpu.VMEM((3 * VP,), jnp.float32),
            pltpu.VMEM((3 * VP,), jnp.float32),
            pltpu.VMEM((3 * VP,), jnp.float32),
            pltpu.VMEM((3, QPW), jnp.float32),
            pltpu.VMEM((QPW,), jnp.int32),
            pltpu.VMEM((QPW,), jnp.float32),
            pltpu.VMEM((QPW,), jnp.int32),
            pltpu.VMEM((QPW,), jnp.float32),
            pltpu.VMEM((4, 16), jnp.float32),
            pltpu.SemaphoreType.DMA,
        ],
    )
    return run(mesh_t.reshape(B, 3 * VP), recon_t.reshape(B, 3 * VP),
               mesh_nT, rec_nT.reshape(B, 3 * VP), ir, dr, ic, dc)


def _fin_body(p_ref, o_ref):
    x = p_ref[0]
    s = jnp.sum(jnp.sum(x, axis=0), axis=1)
    o_ref[0] = s * (1.0 / V)


def _finalize(psums):
    B = psums.shape[0]
    out = pl.pallas_call(
        _fin_body,
        grid=(B,),
        in_specs=[pl.BlockSpec((1, NW, 4, 16), lambda b: (b, 0, 0, 0))],
        out_specs=pl.BlockSpec((None, 1, 4), lambda b: (b, 0, 0)),
        out_shape=jax.ShapeDtypeStruct((B, 1, 4), jnp.float32),
    )(psums)
    return out[:, 0, :3]


@jax.jit
def _impl(joints, meshes, recons, face, recons_faces):
    B = meshes.shape[0]
    pad = ((0, 0), (0, VP - V), (0, 0))
    meshes_p = jnp.pad(meshes, pad, constant_values=PAD_COORD)
    recons_p = jnp.pad(recons, pad, constant_values=PAD_COORD)
    meshes_pT = jnp.transpose(meshes_p, (0, 2, 1))
    recons_pT = jnp.transpose(recons_p, (0, 2, 1))

    facep = jnp.pad(face, ((0, FP - F), (0, 0))).reshape(3 * FP)
    rfacep = jnp.pad(recons_faces,
                     ((0, 0), (0, FP - F), (0, 0))).reshape(B, 3 * FP)
    partials = _normals_partials(
        meshes.reshape(B, 3 * V), recons.reshape(B, 3 * V), facep, rfacep)
    normals = _normalize(partials.reshape(2 * B, NW, 3, VP))
    mesh_nT, rec_nT = normals[:B], normals[B:]

    row_d2, row_idx, col_d2, col_idx = _knn_both(meshes_p, recons_pT)

    psums = _loss_partials(meshes_pT, recons_pT, mesh_nT, rec_nT,
                           row_idx, row_d2, col_idx, col_d2)
    return _finalize(psums)


def kernel(joints, meshes, recons, face, recons_faces):
    return _impl(joints, meshes, recons, face, recons_faces)
